# Initial kernel scaffold; baseline (speedup 1.0000x reference)
import jax
import jax.numpy as jnp
from jax.experimental import pallas as pl


def kernel(x, edge_index, W1, b1, W2, b2, Wfc, bfc):
    raise NotImplementedError("write your pallas kernel here")


# speedup vs baseline: 93.8085x; 93.8085x over previous
import functools

import jax
import jax.numpy as jnp
from jax import lax
from jax.experimental import pallas as pl
from jax.experimental.pallas import tpu as pltpu
from jax.experimental.pallas import tpu_sc as plsc

N = 10000
E = 320000
D = 128
H = 16

NC = 2
NS = 16
NW = NC * NS
CH = 128
KC = 80
E_PAD = NW * CH * KC
EPW = KC * CH
N_PAD = 10112
RPT = N_PAD // NS

_mesh = plsc.VectorSubcoreMesh(core_axis_name="c", subcore_axis_name="s")
_sc_params = pltpu.CompilerParams(use_tc_tiling_on_sc=False)
_sc_params_nl = pltpu.CompilerParams(use_tc_tiling_on_sc=False,
                                     needs_layout_passes=False)


def _wid():
    return lax.axis_index("s") * NC + lax.axis_index("c")


EPT = E // NW
FR = EPT // CH
TAIL = EPT - FR * CH


@functools.partial(
    pl.kernel,
    out_type=(jax.ShapeDtypeStruct((NC * N_PAD, H), jnp.float32),
              jax.ShapeDtypeStruct((NW, KC, CH), jnp.int32),
              jax.ShapeDtypeStruct((NW, KC, CH), jnp.int32)),
    mesh=_mesh,
    compiler_params=_sc_params,
    scratch_types=[
        pltpu.VMEM((KC, CH), jnp.int32),
        pltpu.VMEM((KC, CH), jnp.int32),
        pltpu.VMEM((CH, H), jnp.float32),
        pltpu.VMEM_SHARED((N_PAD, H), jnp.float32),
        pltpu.SemaphoreType.DMA,
        pltpu.SemaphoreType.DMA,
        pltpu.SemaphoreType.DMA,
        pltpu.SemaphoreType.DMA,
        pltpu.SemaphoreType.DMA,
        pltpu.SemaphoreType.DMA,
        pltpu.SemaphoreType.DMA,
        pltpu.SemaphoreType.DMA,
        pltpu.SemaphoreType.DMA,
        pltpu.SemaphoreType.DMA,
        pltpu.SemaphoreType.DMA,
    ],
)
def _sc_count(ei, ones, zeros, out, src3p, dst3p,
              srcb, dstb, rows, acc, sems, semd, sem,
              l0, l1, l2, l3, l4, l5, l6, l7):
    cid = lax.axis_index("c")
    sid = lax.axis_index("s")
    wid = _wid()
    base = wid * EPT
    QD = 8
    ls = (l0, l1, l2, l3, l4, l5, l6, l7)
    pltpu.sync_copy(ones, rows)
    pltpu.sync_copy(zeros.at[pl.ds(sid * RPT, RPT)], acc.at[pl.ds(sid * RPT, RPT)])

    zi = jnp.zeros((16,), jnp.int32)
    ni = jnp.full((16,), N, jnp.int32)
    for t in range(TAIL, CH, 16):
        srcb[FR, pl.ds(t, 16)] = zi
        dstb[FR, pl.ds(t, 16)] = ni
    for jp in range(FR + 1, KC):
        for t in range(0, CH, 16):
            srcb[jp, pl.ds(t, 16)] = zi
            dstb[jp, pl.ds(t, 16)] = ni
    for jp in range(QD):
        pltpu.async_copy(ei.at[1, pl.ds(base + jp * CH, CH)], dstb.at[jp], ls[jp])
    pltpu.async_copy(ei.at[1, pl.ds(base + FR * CH, TAIL)],
                     dstb.at[FR, pl.ds(0, TAIL)], semd)
    plsc.subcore_barrier()

    def outer(jj, _):
        j8 = jj * QD
        for k in range(QD):
            j = j8 + k
            pltpu.make_async_copy(ei.at[1, pl.ds(base, CH)], dstb.at[k],
                                  ls[k]).wait()
            pltpu.async_copy(rows, acc.at[dstb.at[j]], sem, add=True)

            @pl.when(j >= QD)
            def _drain_sc():
                pltpu.make_async_copy(rows, acc.at[dstb.at[j]], sem).wait()

            @pl.when(j + QD < FR)
            def _pf():
                pltpu.async_copy(ei.at[1, pl.ds(base + (j + QD) * CH, CH)],
                                 dstb.at[j + QD], ls[k])
            pltpu.async_copy(ei.at[0, pl.ds(base + j * CH, CH)], srcb.at[j], sems)

            @pl.when(j >= QD)
            def _drain_src():
                pltpu.make_async_copy(ei.at[0, pl.ds(base, CH)], srcb.at[0],
                                      sems).wait()
        return 0

    FR8 = (FR // QD) * QD
    lax.fori_loop(0, FR8 // QD, outer, 0)
    for j in range(FR8, FR):
        k = j % QD
        pltpu.make_async_copy(ei.at[1, pl.ds(base, CH)], dstb.at[k], ls[k]).wait()
        pltpu.async_copy(rows, acc.at[dstb.at[j]], sem, add=True)
        pltpu.make_async_copy(rows, acc.at[dstb.at[j]], sem).wait()
        pltpu.async_copy(ei.at[0, pl.ds(base + j * CH, CH)], srcb.at[j], sems)
        pltpu.make_async_copy(ei.at[0, pl.ds(base, CH)], srcb.at[0], sems).wait()
    pltpu.make_async_copy(ei.at[1, pl.ds(base, TAIL)],
                          dstb.at[FR, pl.ds(0, TAIL)], semd).wait()
    for j in range(FR, KC):
        pltpu.async_copy(rows, acc.at[dstb.at[j]], sem, add=True)
        pltpu.make_async_copy(rows, acc.at[dstb.at[j]], sem).wait()
    pltpu.sync_copy(ei.at[0, pl.ds(base + FR * CH, TAIL)],
                    srcb.at[FR, pl.ds(0, TAIL)])

    def drain_src(j, _):
        pltpu.make_async_copy(ei.at[0, pl.ds(base, CH)], srcb.at[0], sems).wait()
        return 0

    lax.fori_loop(0, QD, drain_src, 0)

    def drain_sc(j, _):
        pltpu.make_async_copy(rows, acc.at[dstb.at[0]], sem).wait()
        return 0

    lax.fori_loop(0, QD, drain_sc, 0)
    pltpu.sync_copy(srcb, src3p.at[wid])
    pltpu.sync_copy(dstb, dst3p.at[wid])
    plsc.subcore_barrier()
    pltpu.sync_copy(acc.at[pl.ds(sid * RPT, RPT)],
                    out.at[pl.ds(cid * N_PAD + sid * RPT, RPT)])


@functools.partial(
    pl.kernel,
    out_type=jax.ShapeDtypeStruct((NC * N_PAD, H), jnp.float32),
    mesh=_mesh,
    compiler_params=_sc_params,
    scratch_types=[
        pltpu.VMEM((KC, CH), jnp.int32),
        pltpu.VMEM((KC, CH), jnp.int32),
        pltpu.VMEM((CH, H), jnp.float32),
        pltpu.VMEM((CH, H), jnp.float32),
        pltpu.VMEM((CH, H), jnp.float32),
        pltpu.VMEM((CH, H), jnp.float32),
        pltpu.VMEM_SHARED((N_PAD, H), jnp.float32),
        pltpu.VMEM_SHARED((N_PAD, H), jnp.float32),
        pltpu.SemaphoreType.DMA,
        pltpu.SemaphoreType.DMA,
        pltpu.SemaphoreType.DMA,
        pltpu.SemaphoreType.DMA,
        pltpu.SemaphoreType.DMA,
        pltpu.SemaphoreType.DMA,
        pltpu.SemaphoreType.DMA,
        pltpu.SemaphoreType.DMA,
    ],
)
def _sc_layer(g, src3, dst3, zeros, out, srcb, dstb, r0, r1, r2, r3,
              tab, acc, g0, g1, g2, g3, s0, s1, s2, s3):
    cid = lax.axis_index("c")
    sid = lax.axis_index("s")
    wid = _wid()
    rows = (r0, r1, r2, r3)
    gs = (g0, g1, g2, g3)
    ss = (s0, s1, s2, s3)
    my = pl.ds(sid * RPT, RPT)
    pltpu.sync_copy(src3.at[wid], srcb)
    pltpu.sync_copy(dst3.at[wid], dstb)
    pltpu.sync_copy(zeros.at[my], acc.at[my])
    pltpu.sync_copy(g.at[my], tab.at[my])
    plsc.subcore_barrier()

    pltpu.async_copy(tab.at[srcb.at[0]], rows[0], gs[0])
    pltpu.async_copy(tab.at[srcb.at[1]], rows[1], gs[1])

    def body(jj, _):
        j = jj * 4
        for b in range(4):
            jc = j + b
            pltpu.make_async_copy(tab.at[srcb.at[jc]], rows[b], gs[b]).wait()
            pltpu.async_copy(rows[b], acc.at[dstb.at[jc]], ss[b], add=True)
            b2 = (b + 2) % 4

            @pl.when(jc + 2 < KC)
            def _prefetch():
                @pl.when(jc >= 2)
                def _wait_prev_scatter():
                    pltpu.make_async_copy(rows[b2], acc.at[dstb.at[jc]],
                                          ss[b2]).wait()
                pltpu.async_copy(tab.at[srcb.at[jc + 2]], rows[b2], gs[b2])
        return 0

    lax.fori_loop(0, KC // 4, body, 0)
    for b in range(4):
        pltpu.make_async_copy(rows[b], acc.at[dstb.at[0]], ss[b]).wait()
    plsc.subcore_barrier()
    pltpu.sync_copy(acc.at[pl.ds(sid * RPT, RPT)],
                    out.at[pl.ds(cid * N_PAD + sid * RPT, RPT)])


@functools.partial(
    pl.kernel,
    out_type=jax.ShapeDtypeStruct((E,), jnp.float32),
    mesh=_mesh,
    compiler_params=_sc_params_nl,
    scratch_types=[
        pltpu.VMEM((2 * N_PAD,), jnp.float32),
        pltpu.VMEM((KC, CH), jnp.int32),
        pltpu.VMEM((KC, CH), jnp.int32),
        pltpu.VMEM((KC, CH), jnp.float32),
        pltpu.SemaphoreType.DMA,
    ],
)
def _sc_edge(pqf, src3, dst3, out, pqv, srcb, dstb, outb, sem):
    wid = _wid()
    base = wid * EPT
    QD = 8
    pltpu.sync_copy(pqf, pqv)
    pltpu.sync_copy(src3.at[wid], srcb)
    pltpu.sync_copy(dst3.at[wid], dstb)

    def body(j, _):
        for k in range(CH // 16):
            si = srcb[j, pl.ds(k * 16, 16)]
            di = dstb[j, pl.ds(k * 16, 16)]
            v = plsc.load_gather(pqv, [si * 2]) + plsc.load_gather(pqv, [di * 2 + 1])
            outb[j, pl.ds(k * 16, 16)] = v

        @pl.when(j < FR)
        def _store():
            pltpu.async_copy(outb.at[j], out.at[pl.ds(base + j * CH, CH)], sem)

        @pl.when(j >= QD)
        def _drain():
            pltpu.make_async_copy(outb.at[0], out.at[pl.ds(base, CH)], sem).wait()
        return 0

    lax.fori_loop(0, FR + 1, body, 0)
    pltpu.async_copy(outb.at[FR, pl.ds(0, TAIL)],
                     out.at[pl.ds(base + FR * CH, TAIL)], sem)

    def drain(j, _):
        pltpu.make_async_copy(outb.at[0], out.at[pl.ds(base, CH)], sem).wait()
        return 0

    lax.fori_loop(0, QD - 1, drain, 0)
    pltpu.make_async_copy(outb.at[0, pl.ds(0, TAIL)],
                          out.at[pl.ds(base, TAIL)], sem).wait()


NB = N_PAD // 8


def _mm1_body(x_ref, w_ref, o_ref):
    o_ref[...] = jnp.dot(x_ref[...], w_ref[...],
                         preferred_element_type=jnp.float32)


def _prep_body(cnt_ref, h1_ref, g1_ref, dinv_ref):
    deg = cnt_ref[0:NB] + cnt_ref[NB:2 * NB] + 1.0
    dinv = lax.rsqrt(deg)
    dinv_ref[...] = dinv
    g1_ref[...] = h1_ref[...] * dinv


def _mid_body(s_ref, g1_ref, dinv_ref, b1_ref, w2_ref, g2_ref):
    dinv = dinv_ref[...]
    agg = dinv * (s_ref[0:NB] + s_ref[NB:2 * NB] + g1_ref[...]) + b1_ref[...]
    h1r = jnp.maximum(agg, 0.0)
    g2_ref[...] = jnp.dot(h1r, w2_ref[...],
                          preferred_element_type=jnp.float32) * dinv


def _post_body(s_ref, g2_ref, dinv_ref, b2_ref, wfc_ref, bfc_ref, pq_ref):
    h2 = dinv_ref[...] * (s_ref[0:NB] + s_ref[NB:2 * NB] + g2_ref[...]) + b2_ref[...]
    pq_ref[...] = jnp.dot(h2, wfc_ref[...],
                          preferred_element_type=jnp.float32) + bfc_ref[...]


_f32 = jnp.float32


def kernel(x, edge_index, W1, b1, W2, b2, Wfc, bfc):
    zeros = jnp.zeros((N_PAD, H), _f32)
    ones = jnp.ones((CH, H), _f32)
    xp = jnp.pad(x, ((0, N_PAD - N), (0, 0)))
    xg = xp.reshape(NB, 8 * D)
    eye8 = jnp.eye(8, dtype=_f32)
    W1blk = jnp.kron(eye8, W1)
    W2blk = jnp.kron(eye8, W2)
    Wfc2 = jnp.concatenate([Wfc[:H], Wfc[H:]], axis=1)
    Wfcblk = jnp.kron(eye8, Wfc2)
    b1p = jnp.tile(b1, 8).reshape(1, 128)
    b2p = jnp.tile(b2, 8).reshape(1, 128)
    bfcp = jnp.tile(jnp.pad(bfc, (0, 1)), 8).reshape(1, 16)

    cnt, src3, dst3 = _sc_count(edge_index, ones, zeros)
    h1p = pl.pallas_call(
        _mm1_body,
        out_shape=jax.ShapeDtypeStruct((NB, 128), _f32),
    )(xg, W1blk)
    g1p, dinvp = pl.pallas_call(
        _prep_body,
        out_shape=(jax.ShapeDtypeStruct((NB, 128), _f32),
                   jax.ShapeDtypeStruct((NB, 128), _f32)),
    )(cnt.reshape(2 * NB, 128), h1p)

    s1 = _sc_layer(g1p.reshape(N_PAD, H), src3, dst3, zeros)

    g2p = pl.pallas_call(
        _mid_body,
        out_shape=jax.ShapeDtypeStruct((NB, 128), _f32),
    )(s1.reshape(2 * NB, 128), g1p, dinvp, b1p, W2blk)

    s2 = _sc_layer(g2p.reshape(N_PAD, H), src3, dst3, zeros)

    pq = pl.pallas_call(
        _post_body,
        out_shape=jax.ShapeDtypeStruct((NB, 16), _f32),
    )(s2.reshape(2 * NB, 128), g2p, dinvp, b2p, Wfcblk, bfcp)

    return _sc_edge(pq.reshape(-1), src3, dst3)


# speedup vs baseline: 95.0081x; 1.0128x over previous
import functools

import jax
import jax.numpy as jnp
from jax import lax
from jax.experimental import pallas as pl
from jax.experimental.pallas import tpu as pltpu
from jax.experimental.pallas import tpu_sc as plsc

N = 10000
E = 320000
D = 128
H = 16

NC = 2
NS = 16
NW = NC * NS
CH = 128
KC = 80
E_PAD = NW * CH * KC
EPW = KC * CH
N_PAD = 10112
RPT = N_PAD // NS

_mesh = plsc.VectorSubcoreMesh(core_axis_name="c", subcore_axis_name="s")
_sc_params = pltpu.CompilerParams(use_tc_tiling_on_sc=False)
_sc_params_nl = pltpu.CompilerParams(use_tc_tiling_on_sc=False,
                                     needs_layout_passes=False)


def _wid():
    return lax.axis_index("s") * NC + lax.axis_index("c")


EPT = E // NW
FR = EPT // CH
TAIL = EPT - FR * CH


@functools.partial(
    pl.kernel,
    out_type=(jax.ShapeDtypeStruct((NC * N_PAD, H), jnp.float32),
              jax.ShapeDtypeStruct((NW, KC, CH), jnp.int32),
              jax.ShapeDtypeStruct((NW, KC, CH), jnp.int32)),
    mesh=_mesh,
    compiler_params=_sc_params,
    scratch_types=[
        pltpu.VMEM((KC, CH), jnp.int32),
        pltpu.VMEM((KC, CH), jnp.int32),
        pltpu.VMEM((CH, H), jnp.float32),
        pltpu.VMEM_SHARED((N_PAD, H), jnp.float32),
        pltpu.SemaphoreType.DMA,
        pltpu.SemaphoreType.DMA,
        pltpu.SemaphoreType.DMA,
        pltpu.SemaphoreType.DMA,
        pltpu.SemaphoreType.DMA,
        pltpu.SemaphoreType.DMA,
        pltpu.SemaphoreType.DMA,
        pltpu.SemaphoreType.DMA,
        pltpu.SemaphoreType.DMA,
        pltpu.SemaphoreType.DMA,
        pltpu.SemaphoreType.DMA,
    ],
)
def _sc_count(ei, ones, zeros, out, src3p, dst3p,
              srcb, dstb, rows, acc, sems, semd, sem,
              l0, l1, l2, l3, l4, l5, l6, l7):
    cid = lax.axis_index("c")
    sid = lax.axis_index("s")
    wid = _wid()
    base = wid * EPT
    QD = 8
    ls = (l0, l1, l2, l3, l4, l5, l6, l7)
    pltpu.sync_copy(ones, rows)
    pltpu.sync_copy(zeros.at[pl.ds(sid * RPT, RPT)], acc.at[pl.ds(sid * RPT, RPT)])

    zi = jnp.zeros((16,), jnp.int32)
    ni = jnp.full((16,), N, jnp.int32)
    for t in range(TAIL, CH, 16):
        srcb[FR, pl.ds(t, 16)] = zi
        dstb[FR, pl.ds(t, 16)] = ni
    for jp in range(FR + 1, KC):
        for t in range(0, CH, 16):
            srcb[jp, pl.ds(t, 16)] = zi
            dstb[jp, pl.ds(t, 16)] = ni
    for jp in range(QD):
        pltpu.async_copy(ei.at[1, pl.ds(base + jp * CH, CH)], dstb.at[jp], ls[jp])
    pltpu.async_copy(ei.at[1, pl.ds(base + FR * CH, TAIL)],
                     dstb.at[FR, pl.ds(0, TAIL)], semd)
    plsc.subcore_barrier()

    def outer(jj, _):
        j8 = jj * QD
        for k in range(QD):
            j = j8 + k
            pltpu.make_async_copy(ei.at[1, pl.ds(base, CH)], dstb.at[k],
                                  ls[k]).wait()
            pltpu.async_copy(rows, acc.at[dstb.at[j]], sem, add=True)

            @pl.when(j >= QD)
            def _drain_sc():
                pltpu.make_async_copy(rows, acc.at[dstb.at[j]], sem).wait()

            @pl.when(j + QD < FR)
            def _pf():
                pltpu.async_copy(ei.at[1, pl.ds(base + (j + QD) * CH, CH)],
                                 dstb.at[j + QD], ls[k])
            pltpu.async_copy(ei.at[0, pl.ds(base + j * CH, CH)], srcb.at[j], sems)

            @pl.when(j >= QD)
            def _drain_src():
                pltpu.make_async_copy(ei.at[0, pl.ds(base, CH)], srcb.at[0],
                                      sems).wait()
        return 0

    FR8 = (FR // QD) * QD
    lax.fori_loop(0, FR8 // QD, outer, 0)
    for j in range(FR8, FR):
        k = j % QD
        pltpu.make_async_copy(ei.at[1, pl.ds(base, CH)], dstb.at[k], ls[k]).wait()
        pltpu.async_copy(rows, acc.at[dstb.at[j]], sem, add=True)
        pltpu.make_async_copy(rows, acc.at[dstb.at[j]], sem).wait()
        pltpu.async_copy(ei.at[0, pl.ds(base + j * CH, CH)], srcb.at[j], sems)
        pltpu.make_async_copy(ei.at[0, pl.ds(base, CH)], srcb.at[0], sems).wait()
    pltpu.make_async_copy(ei.at[1, pl.ds(base, TAIL)],
                          dstb.at[FR, pl.ds(0, TAIL)], semd).wait()
    for j in range(FR, KC):
        pltpu.async_copy(rows, acc.at[dstb.at[j]], sem, add=True)
        pltpu.make_async_copy(rows, acc.at[dstb.at[j]], sem).wait()
    pltpu.sync_copy(ei.at[0, pl.ds(base + FR * CH, TAIL)],
                    srcb.at[FR, pl.ds(0, TAIL)])

    def drain_src(j, _):
        pltpu.make_async_copy(ei.at[0, pl.ds(base, CH)], srcb.at[0], sems).wait()
        return 0

    lax.fori_loop(0, QD, drain_src, 0)

    def drain_sc(j, _):
        pltpu.make_async_copy(rows, acc.at[dstb.at[0]], sem).wait()
        return 0

    lax.fori_loop(0, QD, drain_sc, 0)
    pltpu.sync_copy(srcb, src3p.at[wid])
    pltpu.sync_copy(dstb, dst3p.at[wid])
    plsc.subcore_barrier()
    pltpu.sync_copy(acc.at[pl.ds(sid * RPT, RPT)],
                    out.at[pl.ds(cid * N_PAD + sid * RPT, RPT)])


@functools.partial(
    pl.kernel,
    out_type=jax.ShapeDtypeStruct((NC * N_PAD, H), jnp.float32),
    mesh=_mesh,
    compiler_params=_sc_params,
    scratch_types=[
        pltpu.VMEM((KC, CH), jnp.int32),
        pltpu.VMEM((KC, CH), jnp.int32),
        pltpu.VMEM((CH, H), jnp.float32),
        pltpu.VMEM((CH, H), jnp.float32),
        pltpu.VMEM((CH, H), jnp.float32),
        pltpu.VMEM((CH, H), jnp.float32),
        pltpu.VMEM((CH, H), jnp.float32),
        pltpu.VMEM((CH, H), jnp.float32),
        pltpu.VMEM((CH, H), jnp.float32),
        pltpu.VMEM((CH, H), jnp.float32),
        pltpu.VMEM_SHARED((N_PAD, H), jnp.float32),
        pltpu.VMEM_SHARED((N_PAD, H), jnp.float32),
        pltpu.SemaphoreType.DMA,
        pltpu.SemaphoreType.DMA,
        pltpu.SemaphoreType.DMA,
        pltpu.SemaphoreType.DMA,
        pltpu.SemaphoreType.DMA,
        pltpu.SemaphoreType.DMA,
        pltpu.SemaphoreType.DMA,
        pltpu.SemaphoreType.DMA,
        pltpu.SemaphoreType.DMA,
        pltpu.SemaphoreType.DMA,
        pltpu.SemaphoreType.DMA,
        pltpu.SemaphoreType.DMA,
        pltpu.SemaphoreType.DMA,
        pltpu.SemaphoreType.DMA,
        pltpu.SemaphoreType.DMA,
        pltpu.SemaphoreType.DMA,
    ],
)
def _sc_layer(g, src3, dst3, zeros, out, srcb, dstb,
              r0, r1, r2, r3, r4, r5, r6, r7, tab, acc,
              g0, g1, g2, g3, g4, g5, g6, g7,
              s0, s1, s2, s3, s4, s5, s6, s7):
    cid = lax.axis_index("c")
    sid = lax.axis_index("s")
    wid = _wid()
    rows = (r0, r1, r2, r3, r4, r5, r6, r7)
    gs = (g0, g1, g2, g3, g4, g5, g6, g7)
    ss = (s0, s1, s2, s3, s4, s5, s6, s7)
    NBUF = 8
    DP = NBUF // 2
    my = pl.ds(sid * RPT, RPT)
    pltpu.sync_copy(src3.at[wid], srcb)
    pltpu.sync_copy(dst3.at[wid], dstb)
    pltpu.sync_copy(zeros.at[my], acc.at[my])
    pltpu.sync_copy(g.at[my], tab.at[my])
    plsc.subcore_barrier()

    for b in range(DP):
        pltpu.async_copy(tab.at[srcb.at[b]], rows[b], gs[b])

    def body(jj, _):
        j = jj * NBUF
        for b in range(NBUF):
            jc = j + b
            pltpu.make_async_copy(tab.at[srcb.at[jc]], rows[b], gs[b]).wait()
            pltpu.async_copy(rows[b], acc.at[dstb.at[jc]], ss[b], add=True)
            b2 = (b + DP) % NBUF

            @pl.when(jc + DP < KC)
            def _prefetch():
                @pl.when(jc >= DP)
                def _wait_prev_scatter():
                    pltpu.make_async_copy(rows[b2], acc.at[dstb.at[jc]],
                                          ss[b2]).wait()
                pltpu.async_copy(tab.at[srcb.at[jc + DP]], rows[b2], gs[b2])
        return 0

    lax.fori_loop(0, KC // NBUF, body, 0)
    for b in range(NBUF):
        pltpu.make_async_copy(rows[b], acc.at[dstb.at[0]], ss[b]).wait()
    plsc.subcore_barrier()
    pltpu.sync_copy(acc.at[pl.ds(sid * RPT, RPT)],
                    out.at[pl.ds(cid * N_PAD + sid * RPT, RPT)])


@functools.partial(
    pl.kernel,
    out_type=jax.ShapeDtypeStruct((E,), jnp.float32),
    mesh=_mesh,
    compiler_params=_sc_params_nl,
    scratch_types=[
        pltpu.VMEM((2 * N_PAD,), jnp.float32),
        pltpu.VMEM((KC, CH), jnp.int32),
        pltpu.VMEM((KC, CH), jnp.int32),
        pltpu.VMEM((KC, CH), jnp.float32),
        pltpu.SemaphoreType.DMA,
    ],
)
def _sc_edge(pqf, src3, dst3, out, pqv, srcb, dstb, outb, sem):
    wid = _wid()
    base = wid * EPT
    QD = 8
    pltpu.sync_copy(pqf, pqv)
    pltpu.sync_copy(src3.at[wid], srcb)
    pltpu.sync_copy(dst3.at[wid], dstb)

    def body(j, _):
        for k in range(CH // 16):
            si = srcb[j, pl.ds(k * 16, 16)]
            di = dstb[j, pl.ds(k * 16, 16)]
            v = plsc.load_gather(pqv, [si * 2]) + plsc.load_gather(pqv, [di * 2 + 1])
            outb[j, pl.ds(k * 16, 16)] = v

        @pl.when(j < FR)
        def _store():
            pltpu.async_copy(outb.at[j], out.at[pl.ds(base + j * CH, CH)], sem)

        @pl.when(j >= QD)
        def _drain():
            pltpu.make_async_copy(outb.at[0], out.at[pl.ds(base, CH)], sem).wait()
        return 0

    lax.fori_loop(0, FR + 1, body, 0)
    pltpu.async_copy(outb.at[FR, pl.ds(0, TAIL)],
                     out.at[pl.ds(base + FR * CH, TAIL)], sem)

    def drain(j, _):
        pltpu.make_async_copy(outb.at[0], out.at[pl.ds(base, CH)], sem).wait()
        return 0

    lax.fori_loop(0, QD - 1, drain, 0)
    pltpu.make_async_copy(outb.at[0, pl.ds(0, TAIL)],
                          out.at[pl.ds(base, TAIL)], sem).wait()


NB = N_PAD // 8


def _mm1_body(x_ref, w_ref, o_ref):
    o_ref[...] = jnp.dot(x_ref[...], w_ref[...],
                         preferred_element_type=jnp.float32)


def _prep_body(cnt_ref, h1_ref, g1_ref, dinv_ref):
    deg = cnt_ref[0:NB] + cnt_ref[NB:2 * NB] + 1.0
    dinv = lax.rsqrt(deg)
    dinv_ref[...] = dinv
    g1_ref[...] = h1_ref[...] * dinv


def _mid_body(s_ref, g1_ref, dinv_ref, b1_ref, w2_ref, g2_ref):
    dinv = dinv_ref[...]
    agg = dinv * (s_ref[0:NB] + s_ref[NB:2 * NB] + g1_ref[...]) + b1_ref[...]
    h1r = jnp.maximum(agg, 0.0)
    g2_ref[...] = jnp.dot(h1r, w2_ref[...],
                          preferred_element_type=jnp.float32) * dinv


def _post_body(s_ref, g2_ref, dinv_ref, b2_ref, wfc_ref, bfc_ref, pq_ref):
    h2 = dinv_ref[...] * (s_ref[0:NB] + s_ref[NB:2 * NB] + g2_ref[...]) + b2_ref[...]
    pq_ref[...] = jnp.dot(h2, wfc_ref[...],
                          preferred_element_type=jnp.float32) + bfc_ref[...]


_f32 = jnp.float32


def kernel(x, edge_index, W1, b1, W2, b2, Wfc, bfc):
    zeros = jnp.zeros((N_PAD, H), _f32)
    ones = jnp.ones((CH, H), _f32)
    xp = jnp.pad(x, ((0, N_PAD - N), (0, 0)))
    xg = xp.reshape(NB, 8 * D)
    eye8 = jnp.eye(8, dtype=_f32)
    W1blk = jnp.kron(eye8, W1)
    W2blk = jnp.kron(eye8, W2)
    Wfc2 = jnp.concatenate([Wfc[:H], Wfc[H:]], axis=1)
    Wfcblk = jnp.kron(eye8, Wfc2)
    b1p = jnp.tile(b1, 8).reshape(1, 128)
    b2p = jnp.tile(b2, 8).reshape(1, 128)
    bfcp = jnp.tile(jnp.pad(bfc, (0, 1)), 8).reshape(1, 16)

    cnt, src3, dst3 = _sc_count(edge_index, ones, zeros)
    h1p = pl.pallas_call(
        _mm1_body,
        out_shape=jax.ShapeDtypeStruct((NB, 128), _f32),
    )(xg, W1blk)
    g1p, dinvp = pl.pallas_call(
        _prep_body,
        out_shape=(jax.ShapeDtypeStruct((NB, 128), _f32),
                   jax.ShapeDtypeStruct((NB, 128), _f32)),
    )(cnt.reshape(2 * NB, 128), h1p)

    s1 = _sc_layer(g1p.reshape(N_PAD, H), src3, dst3, zeros)

    g2p = pl.pallas_call(
        _mid_body,
        out_shape=jax.ShapeDtypeStruct((NB, 128), _f32),
    )(s1.reshape(2 * NB, 128), g1p, dinvp, b1p, W2blk)

    s2 = _sc_layer(g2p.reshape(N_PAD, H), src3, dst3, zeros)

    pq = pl.pallas_call(
        _post_body,
        out_shape=jax.ShapeDtypeStruct((NB, 16), _f32),
    )(s2.reshape(2 * NB, 128), g2p, dinvp, b2p, Wfcblk, bfcp)

    return _sc_edge(pq.reshape(-1), src3, dst3)


# speedup vs baseline: 95.4054x; 1.0042x over previous
import functools

import jax
import jax.numpy as jnp
from jax import lax
from jax.experimental import pallas as pl
from jax.experimental.pallas import tpu as pltpu
from jax.experimental.pallas import tpu_sc as plsc

N = 10000
E = 320000
D = 128
H = 16

NC = 2
NS = 16
NW = NC * NS
CH = 128
KC = 80
E_PAD = NW * CH * KC
EPW = KC * CH
N_PAD = 10112
RPT = N_PAD // NS

_mesh = plsc.VectorSubcoreMesh(core_axis_name="c", subcore_axis_name="s")
_sc_params = pltpu.CompilerParams(use_tc_tiling_on_sc=False)
_sc_params_nl = pltpu.CompilerParams(use_tc_tiling_on_sc=False,
                                     needs_layout_passes=False)


def _wid():
    return lax.axis_index("s") * NC + lax.axis_index("c")


EPT = E // NW
FR = EPT // CH
TAIL = EPT - FR * CH


@functools.partial(
    pl.kernel,
    out_type=(jax.ShapeDtypeStruct((NC * N_PAD, H), jnp.float32),
              jax.ShapeDtypeStruct((NW, KC, CH), jnp.int32),
              jax.ShapeDtypeStruct((NW, KC, CH), jnp.int32)),
    mesh=_mesh,
    compiler_params=_sc_params,
    scratch_types=[
        pltpu.VMEM((KC, CH), jnp.int32),
        pltpu.VMEM((KC, CH), jnp.int32),
        pltpu.VMEM((CH, H), jnp.float32),
        pltpu.VMEM_SHARED((N_PAD, H), jnp.float32),
        pltpu.SemaphoreType.DMA,
        pltpu.SemaphoreType.DMA,
        pltpu.SemaphoreType.DMA,
        pltpu.SemaphoreType.DMA,
        pltpu.SemaphoreType.DMA,
        pltpu.SemaphoreType.DMA,
        pltpu.SemaphoreType.DMA,
        pltpu.SemaphoreType.DMA,
        pltpu.SemaphoreType.DMA,
        pltpu.SemaphoreType.DMA,
        pltpu.SemaphoreType.DMA,
    ],
)
def _sc_count(ei, ones, zeros, out, src3p, dst3p,
              srcb, dstb, rows, acc, sems, semd, sem,
              l0, l1, l2, l3, l4, l5, l6, l7):
    cid = lax.axis_index("c")
    sid = lax.axis_index("s")
    wid = _wid()
    base = wid * EPT
    QD = 8
    ls = (l0, l1, l2, l3, l4, l5, l6, l7)
    pltpu.sync_copy(ones, rows)
    pltpu.sync_copy(zeros.at[pl.ds(sid * RPT, RPT)], acc.at[pl.ds(sid * RPT, RPT)])

    zi = jnp.zeros((16,), jnp.int32)
    ni = jnp.full((16,), N, jnp.int32)
    for t in range(TAIL, CH, 16):
        srcb[FR, pl.ds(t, 16)] = zi
        dstb[FR, pl.ds(t, 16)] = ni
    for jp in range(FR + 1, KC):
        for t in range(0, CH, 16):
            srcb[jp, pl.ds(t, 16)] = zi
            dstb[jp, pl.ds(t, 16)] = ni
    for jp in range(QD):
        pltpu.async_copy(ei.at[1, pl.ds(base + jp * CH, CH)], dstb.at[jp], ls[jp])
    pltpu.async_copy(ei.at[1, pl.ds(base + FR * CH, TAIL)],
                     dstb.at[FR, pl.ds(0, TAIL)], semd)
    plsc.subcore_barrier()

    def outer(jj, _):
        j8 = jj * QD
        for k in range(QD):
            j = j8 + k
            pltpu.make_async_copy(ei.at[1, pl.ds(base, CH)], dstb.at[k],
                                  ls[k]).wait()
            pltpu.async_copy(rows, acc.at[dstb.at[j]], sem, add=True)

            @pl.when(j >= QD)
            def _drain_sc():
                pltpu.make_async_copy(rows, acc.at[dstb.at[j]], sem).wait()

            @pl.when(j + QD < FR)
            def _pf():
                pltpu.async_copy(ei.at[1, pl.ds(base + (j + QD) * CH, CH)],
                                 dstb.at[j + QD], ls[k])
            pltpu.async_copy(ei.at[0, pl.ds(base + j * CH, CH)], srcb.at[j], sems)

            @pl.when(j >= QD)
            def _drain_src():
                pltpu.make_async_copy(ei.at[0, pl.ds(base, CH)], srcb.at[0],
                                      sems).wait()
        return 0

    FR8 = (FR // QD) * QD
    lax.fori_loop(0, FR8 // QD, outer, 0)
    for j in range(FR8, FR):
        k = j % QD
        pltpu.make_async_copy(ei.at[1, pl.ds(base, CH)], dstb.at[k], ls[k]).wait()
        pltpu.async_copy(rows, acc.at[dstb.at[j]], sem, add=True)
        pltpu.make_async_copy(rows, acc.at[dstb.at[j]], sem).wait()
        pltpu.async_copy(ei.at[0, pl.ds(base + j * CH, CH)], srcb.at[j], sems)
        pltpu.make_async_copy(ei.at[0, pl.ds(base, CH)], srcb.at[0], sems).wait()
    pltpu.make_async_copy(ei.at[1, pl.ds(base, TAIL)],
                          dstb.at[FR, pl.ds(0, TAIL)], semd).wait()
    for j in range(FR, KC):
        pltpu.async_copy(rows, acc.at[dstb.at[j]], sem, add=True)
        pltpu.make_async_copy(rows, acc.at[dstb.at[j]], sem).wait()
    pltpu.sync_copy(ei.at[0, pl.ds(base + FR * CH, TAIL)],
                    srcb.at[FR, pl.ds(0, TAIL)])

    def drain_src(j, _):
        pltpu.make_async_copy(ei.at[0, pl.ds(base, CH)], srcb.at[0], sems).wait()
        return 0

    lax.fori_loop(0, QD, drain_src, 0)

    def drain_sc(j, _):
        pltpu.make_async_copy(rows, acc.at[dstb.at[0]], sem).wait()
        return 0

    lax.fori_loop(0, QD, drain_sc, 0)
    pltpu.sync_copy(srcb, src3p.at[wid])
    pltpu.sync_copy(dstb, dst3p.at[wid])
    plsc.subcore_barrier()
    pltpu.sync_copy(acc.at[pl.ds(sid * RPT, RPT)],
                    out.at[pl.ds(cid * N_PAD + sid * RPT, RPT)])


@functools.partial(
    pl.kernel,
    out_type=jax.ShapeDtypeStruct((NC * N_PAD, H), jnp.float32),
    mesh=_mesh,
    compiler_params=_sc_params,
    scratch_types=[
        pltpu.VMEM((KC, CH), jnp.int32),
        pltpu.VMEM((KC, CH), jnp.int32),
        pltpu.VMEM((CH, H), jnp.float32),
        pltpu.VMEM((CH, H), jnp.float32),
        pltpu.VMEM((CH, H), jnp.float32),
        pltpu.VMEM((CH, H), jnp.float32),
        pltpu.VMEM((CH, H), jnp.float32),
        pltpu.VMEM((CH, H), jnp.float32),
        pltpu.VMEM((CH, H), jnp.float32),
        pltpu.VMEM((CH, H), jnp.float32),
        pltpu.VMEM_SHARED((N_PAD, H), jnp.float32),
        pltpu.VMEM_SHARED((N_PAD, H), jnp.float32),
        pltpu.SemaphoreType.DMA,
        pltpu.SemaphoreType.DMA,
        pltpu.SemaphoreType.DMA,
        pltpu.SemaphoreType.DMA,
        pltpu.SemaphoreType.DMA,
        pltpu.SemaphoreType.DMA,
        pltpu.SemaphoreType.DMA,
        pltpu.SemaphoreType.DMA,
        pltpu.SemaphoreType.DMA,
        pltpu.SemaphoreType.DMA,
        pltpu.SemaphoreType.DMA,
        pltpu.SemaphoreType.DMA,
        pltpu.SemaphoreType.DMA,
        pltpu.SemaphoreType.DMA,
        pltpu.SemaphoreType.DMA,
        pltpu.SemaphoreType.DMA,
    ],
)
def _sc_layer(g, src3, dst3, zeros, out, srcb, dstb,
              r0, r1, r2, r3, r4, r5, r6, r7, tab, acc,
              g0, g1, g2, g3, g4, g5, g6, g7,
              s0, s1, s2, s3, s4, s5, s6, s7):
    cid = lax.axis_index("c")
    sid = lax.axis_index("s")
    wid = _wid()
    rows = (r0, r1, r2, r3, r4, r5, r6, r7)
    gs = (g0, g1, g2, g3, g4, g5, g6, g7)
    ss = (s0, s1, s2, s3, s4, s5, s6, s7)
    NBUF = 8
    DP = NBUF // 2
    my = pl.ds(sid * RPT, RPT)
    pltpu.sync_copy(src3.at[wid], srcb)
    pltpu.sync_copy(dst3.at[wid], dstb)
    pltpu.sync_copy(zeros.at[my], acc.at[my])
    pltpu.sync_copy(g.at[my], tab.at[my])
    plsc.subcore_barrier()

    for b in range(DP):
        pltpu.async_copy(tab.at[srcb.at[b]], rows[b], gs[b])

    def body(jj, _):
        j = jj * NBUF
        for b in range(NBUF):
            jc = j + b
            pltpu.make_async_copy(tab.at[srcb.at[jc]], rows[b], gs[b]).wait()
            pltpu.async_copy(rows[b], acc.at[dstb.at[jc]], ss[b], add=True)
            b2 = (b + DP) % NBUF

            @pl.when(jc + DP < KC)
            def _prefetch():
                @pl.when(jc >= DP)
                def _wait_prev_scatter():
                    pltpu.make_async_copy(rows[b2], acc.at[dstb.at[jc]],
                                          ss[b2]).wait()
                pltpu.async_copy(tab.at[srcb.at[jc + DP]], rows[b2], gs[b2])
        return 0

    lax.fori_loop(0, KC // NBUF, body, 0)
    for b in range(NBUF):
        pltpu.make_async_copy(rows[b], acc.at[dstb.at[0]], ss[b]).wait()
    plsc.subcore_barrier()
    pltpu.sync_copy(acc.at[pl.ds(sid * RPT, RPT)],
                    out.at[pl.ds(cid * N_PAD + sid * RPT, RPT)])


@functools.partial(
    pl.kernel,
    out_type=jax.ShapeDtypeStruct((E,), jnp.float32),
    mesh=_mesh,
    compiler_params=_sc_params_nl,
    scratch_types=[
        pltpu.VMEM((2 * N_PAD,), jnp.float32),
        pltpu.VMEM((KC, CH), jnp.int32),
        pltpu.VMEM((KC, CH), jnp.int32),
        pltpu.VMEM((KC, CH), jnp.float32),
        pltpu.SemaphoreType.DMA,
    ],
)
def _sc_edge(pqf, src3, dst3, out, pqv, srcb, dstb, outb, sem):
    wid = _wid()
    base = wid * EPT
    QD = 8
    pltpu.sync_copy(pqf, pqv)
    pltpu.sync_copy(src3.at[wid], srcb)
    pltpu.sync_copy(dst3.at[wid], dstb)

    def body(jj, _):
        for t in range(2):
            j = jj * 2 + t
            for k in range(CH // 16):
                si = srcb[j, pl.ds(k * 16, 16)]
                di = dstb[j, pl.ds(k * 16, 16)]
                v = (plsc.load_gather(pqv, [si * 2])
                     + plsc.load_gather(pqv, [di * 2 + 1]))
                outb[j, pl.ds(k * 16, 16)] = v
            pltpu.async_copy(outb.at[j], out.at[pl.ds(base + j * CH, CH)], sem)

            @pl.when(j >= QD)
            def _drain():
                pltpu.make_async_copy(outb.at[0], out.at[pl.ds(base, CH)],
                                      sem).wait()
        return 0

    lax.fori_loop(0, FR // 2, body, 0)
    for k in range(TAIL // 16):
        si = srcb[FR, pl.ds(k * 16, 16)]
        di = dstb[FR, pl.ds(k * 16, 16)]
        outb[FR, pl.ds(k * 16, 16)] = (plsc.load_gather(pqv, [si * 2])
                                       + plsc.load_gather(pqv, [di * 2 + 1]))
    pltpu.async_copy(outb.at[FR, pl.ds(0, TAIL)],
                     out.at[pl.ds(base + FR * CH, TAIL)], sem)

    def drain(j, _):
        pltpu.make_async_copy(outb.at[0], out.at[pl.ds(base, CH)], sem).wait()
        return 0

    lax.fori_loop(0, QD, drain, 0)
    pltpu.make_async_copy(outb.at[0, pl.ds(0, TAIL)],
                          out.at[pl.ds(base, TAIL)], sem).wait()


NB = N_PAD // 8


def _mm1_body(x_ref, w_ref, o_ref):
    o_ref[...] = jnp.dot(x_ref[...], w_ref[...],
                         preferred_element_type=jnp.float32)


def _prep_body(cnt_ref, h1_ref, g1_ref, dinv_ref):
    deg = cnt_ref[0:NB] + cnt_ref[NB:2 * NB] + 1.0
    dinv = lax.rsqrt(deg)
    dinv_ref[...] = dinv
    g1_ref[...] = h1_ref[...] * dinv


def _mid_body(s_ref, g1_ref, dinv_ref, b1_ref, w2_ref, g2_ref):
    dinv = dinv_ref[...]
    agg = dinv * (s_ref[0:NB] + s_ref[NB:2 * NB] + g1_ref[...]) + b1_ref[...]
    h1r = jnp.maximum(agg, 0.0)
    g2_ref[...] = jnp.dot(h1r, w2_ref[...],
                          preferred_element_type=jnp.float32) * dinv


def _post_body(s_ref, g2_ref, dinv_ref, b2_ref, wfc_ref, bfc_ref, pq_ref):
    h2 = dinv_ref[...] * (s_ref[0:NB] + s_ref[NB:2 * NB] + g2_ref[...]) + b2_ref[...]
    pq_ref[...] = jnp.dot(h2, wfc_ref[...],
                          preferred_element_type=jnp.float32) + bfc_ref[...]


_f32 = jnp.float32


def kernel(x, edge_index, W1, b1, W2, b2, Wfc, bfc):
    zeros = jnp.zeros((N_PAD, H), _f32)
    ones = jnp.ones((CH, H), _f32)
    xp = jnp.pad(x, ((0, N_PAD - N), (0, 0)))
    xg = xp.reshape(NB, 8 * D)
    eye8 = jnp.eye(8, dtype=_f32)
    W1blk = jnp.kron(eye8, W1)
    W2blk = jnp.kron(eye8, W2)
    Wfc2 = jnp.concatenate([Wfc[:H], Wfc[H:]], axis=1)
    Wfcblk = jnp.kron(eye8, Wfc2)
    b1p = jnp.tile(b1, 8).reshape(1, 128)
    b2p = jnp.tile(b2, 8).reshape(1, 128)
    bfcp = jnp.tile(jnp.pad(bfc, (0, 1)), 8).reshape(1, 16)

    cnt, src3, dst3 = _sc_count(edge_index, ones, zeros)
    h1p = pl.pallas_call(
        _mm1_body,
        out_shape=jax.ShapeDtypeStruct((NB, 128), _f32),
    )(xg, W1blk)
    g1p, dinvp = pl.pallas_call(
        _prep_body,
        out_shape=(jax.ShapeDtypeStruct((NB, 128), _f32),
                   jax.ShapeDtypeStruct((NB, 128), _f32)),
    )(cnt.reshape(2 * NB, 128), h1p)

    s1 = _sc_layer(g1p.reshape(N_PAD, H), src3, dst3, zeros)

    g2p = pl.pallas_call(
        _mid_body,
        out_shape=jax.ShapeDtypeStruct((NB, 128), _f32),
    )(s1.reshape(2 * NB, 128), g1p, dinvp, b1p, W2blk)

    s2 = _sc_layer(g2p.reshape(N_PAD, H), src3, dst3, zeros)

    pq = pl.pallas_call(
        _post_body,
        out_shape=jax.ShapeDtypeStruct((NB, 16), _f32),
    )(s2.reshape(2 * NB, 128), g2p, dinvp, b2p, Wfcblk, bfcp)

    return _sc_edge(pq.reshape(-1), src3, dst3)
